# baseline (speedup 1.0000x reference)
import functools

import jax
import jax.numpy as jnp
from jax import lax
from jax.experimental import pallas as pl
from jax.experimental.pallas import tpu as pltpu
from jax.experimental.pallas import tpu_sc as plsc

VOCAB = 1000000
DIM = 64
SEQ = 50
BATCH = 16384
ROWS = BATCH * SEQ
NUM_WORKERS = 32
PER_W = ROWS // NUM_WORKERS
JBLK = 4
NCOL = SEQ * JBLK

_mesh = plsc.VectorSubcoreMesh(core_axis_name="c", subcore_axis_name="s")


@functools.partial(
    pl.kernel,
    mesh=_mesh,
    out_type=jax.ShapeDtypeStruct((SEQ, 8, 128, 8, 128), jnp.float32),
    scratch_types=[
        pltpu.VMEM((PER_W,), jnp.int32),
        pltpu.VMEM((2, 128), jnp.int32),
        pltpu.VMEM((2, 128, DIM), jnp.float32),
        pltpu.VMEM((2, 1, 8, 1, 8, 128), jnp.float32),
        pltpu.SemaphoreType.DMA,
        pltpu.SemaphoreType.DMA,
        pltpu.SemaphoreType.DMA,
        pltpu.SemaphoreType.DMA,
    ],
    compiler_params=pltpu.CompilerParams(
        use_tc_tiling_on_sc=False, needs_layout_passes=False),
)
def _gather(idx_hbm, table_hbm, out_hbm, idx_v, colidx, rows_v, tbuf,
            g0, g1, w0, w1):
    gsem = (g0, g1)
    wsem = (w0, w1)
    wid = lax.axis_index("s") * 2 + lax.axis_index("c")
    base = wid * PER_W
    pltpu.sync_copy(idx_hbm.at[pl.ds(base, PER_W)], idx_v)

    iota = lax.iota(jnp.int32, 16)

    def compact_col(n, p):
        jb = n // SEQ
        s = n - jb * SEQ
        colbase = jb * (128 * SEQ) + s
        for k in range(8):
            pos = iota * SEQ + (colbase + k * 16 * SEQ)
            vals = plsc.load_gather(idx_v, [pos])
            colidx[p, pl.ds(k * 16, 16)] = vals

    def in_copy(p):
        return pltpu.make_async_copy(
            table_hbm.at[colidx.at[p]], rows_v.at[p], gsem[p])

    def transpose_col(p):
        for f in range(DIM):
            for k in range(8):
                src = plsc.load_gather(
                    rows_v.at[p], [iota + k * 16, jnp.full((16,), f, jnp.int32)])
                tbuf[p, 0, f // 8, 0, f % 8, pl.ds(k * 16, 16)] = src

    def out_copy(n, p):
        jb = n // SEQ
        s = n - jb * SEQ
        j = wid * JBLK + jb
        return pltpu.make_async_copy(
            tbuf.at[p],
            out_hbm.at[pl.ds(s, 1), pl.ds(0, 8), pl.ds(j, 1)], wsem[p])

    compact_col(0, 0)
    in_copy(0).start()
    compact_col(1, 1)
    in_copy(1).start()

    def body(n, carry):
        for p in range(2):
            m = n * 2 + p
            in_copy(p).wait()
            transpose_col(p)
            out_copy(m, p).start()
            out_copy(m, p).wait()
            compact_col(m + 2, p)
            in_copy(p).start()
        return carry

    lax.fori_loop(0, NCOL // 2 - 1, body, 0)

    for p in range(2):
        m = NCOL - 2 + p
        in_copy(p).wait()
        transpose_col(p)
        out_copy(m, p).start()
    for p in range(2):
        out_copy(NCOL - 2 + p, p).wait()


def kernel(x, weight):
    idx = x.reshape(ROWS)
    out = _gather(idx, weight)
    return out.transpose(2, 4, 0, 1, 3).reshape(BATCH, SEQ, DIM)


# speedup vs baseline: 1.6597x; 1.6597x over previous
import functools

import jax
import jax.numpy as jnp
from jax import lax
from jax.experimental import pallas as pl
from jax.experimental.pallas import tpu as pltpu
from jax.experimental.pallas import tpu_sc as plsc

VOCAB = 1000000
DIM = 64
ROWS = 16384 * 50
NUM_WORKERS = 32
PER_W = ROWS // NUM_WORKERS
NBUF = 2
CHUNK = 640
NCH = PER_W // CHUNK
NOUT = NCH // NBUF

_mesh = plsc.VectorSubcoreMesh(core_axis_name="c", subcore_axis_name="s")


@functools.partial(
    pl.kernel,
    mesh=_mesh,
    out_type=jax.ShapeDtypeStruct((ROWS, DIM), jnp.float32),
    scratch_types=[
        pltpu.VMEM((PER_W,), jnp.int32),
        pltpu.VMEM((NBUF, CHUNK, DIM), jnp.float32),
        pltpu.SemaphoreType.DMA,
        pltpu.SemaphoreType.DMA,
        pltpu.SemaphoreType.DMA,
        pltpu.SemaphoreType.DMA,
    ],
    compiler_params=pltpu.CompilerParams(use_tc_tiling_on_sc=False),
)
def _gather(idx_hbm, table_hbm, out_hbm, idx_v, rows_v, g0, g1, w0, w1):
    gsem = (g0, g1)
    wsem = (w0, w1)
    wid = lax.axis_index("s") * 2 + lax.axis_index("c")
    base = wid * PER_W
    pltpu.sync_copy(idx_hbm.at[pl.ds(base, PER_W)], idx_v)

    def in_copy(off, b):
        return pltpu.make_async_copy(
            table_hbm.at[idx_v.at[pl.ds(off, CHUNK)]], rows_v.at[b], gsem[b])

    def out_copy(off, b):
        return pltpu.make_async_copy(
            rows_v.at[b], out_hbm.at[pl.ds(base + off, CHUNK)], wsem[b])

    for b in range(NBUF):
        in_copy(b * CHUNK, b).start()

    def body(g, carry):
        for b in range(NBUF):
            off = pl.multiple_of((g * NBUF + b) * CHUNK, CHUNK)
            in_copy(off, b).wait()
            out_copy(off, b).start()
            out_copy(off, b).wait()
            in_copy(off + NBUF * CHUNK, b).start()
        return carry

    lax.fori_loop(0, NOUT - 1, body, 0)

    for b in range(NBUF):
        off = ((NOUT - 1) * NBUF + b) * CHUNK
        in_copy(off, b).wait()
        out_copy(off, b).start()
    for b in range(NBUF):
        off = ((NOUT - 1) * NBUF + b) * CHUNK
        out_copy(off, b).wait()


def kernel(x, weight):
    idx = x.reshape(ROWS)
    out = _gather(idx, weight)
    return out.reshape(16384, 50, DIM)
